# Initial kernel scaffold; baseline (speedup 1.0000x reference)
import jax
import jax.numpy as jnp
from jax.experimental import pallas as pl


def kernel(gene_pos, pe):
    raise NotImplementedError("write your pallas kernel here")


# speedup vs baseline: 3.4283x; 3.4283x over previous
import functools

import jax
import jax.numpy as jnp
from jax import lax
from jax.experimental import pallas as pl
from jax.experimental.pallas import tpu as pltpu
from jax.experimental.pallas import tpu_sc as plsc

D_MODEL = 64
CHUNK = 128


@functools.lru_cache(maxsize=None)
def _build_sc_gather(n_total: int):
    info = plsc.get_sparse_core_info()
    num_workers = info.num_cores * info.num_subcores
    per_w = n_total // num_workers
    assert per_w * num_workers == n_total and per_w % CHUNK == 0
    n_chunks = per_w // CHUNK

    mesh = plsc.VectorSubcoreMesh(core_axis_name="c", subcore_axis_name="s")

    @functools.partial(
        pl.kernel,
        mesh=mesh,
        out_type=jax.ShapeDtypeStruct((n_total, D_MODEL), jnp.float32),
        scratch_types=[
            pltpu.VMEM((n_chunks, CHUNK), jnp.int32),
            pltpu.VMEM((CHUNK, D_MODEL), jnp.float32),
            pltpu.SemaphoreType.DMA,
        ],
        compiler_params=pltpu.CompilerParams(use_tc_tiling_on_sc=False),
    )
    def gather_kernel(table_hbm, idx_hbm, out_hbm, idx_v, rows_v, sem):
        wid = lax.axis_index("s") * info.num_cores + lax.axis_index("c")
        base = wid * per_w
        pltpu.sync_copy(idx_hbm.at[pl.ds(wid * n_chunks, n_chunks)], idx_v)

        def body(j, carry):
            pltpu.async_copy(table_hbm.at[idx_v.at[j]], rows_v, sem).wait()
            pltpu.sync_copy(rows_v, out_hbm.at[pl.ds(base + j * CHUNK, CHUNK)])
            return carry

        lax.fori_loop(0, n_chunks, body, 0)

    return gather_kernel


def kernel(gene_pos, pe):
    b, s = gene_pos.shape
    n_total = b * s
    idx = gene_pos.reshape(n_total // CHUNK, CHUNK)
    table = pe.reshape(pe.shape[0], D_MODEL)
    out = _build_sc_gather(n_total)(table, idx)
    return out.reshape(b, s, D_MODEL)


# speedup vs baseline: 3.6059x; 1.0518x over previous
import functools

import jax
import jax.numpy as jnp
from jax import lax
from jax.experimental import pallas as pl
from jax.experimental.pallas import tpu as pltpu
from jax.experimental.pallas import tpu_sc as plsc

D_MODEL = 64
CHUNK = 128
GROUP = 4
NBUF = 2


@functools.lru_cache(maxsize=None)
def _build_sc_gather(n_total: int):
    info = plsc.get_sparse_core_info()
    num_workers = info.num_cores * info.num_subcores
    per_w = n_total // num_workers
    assert per_w * num_workers == n_total and per_w % (CHUNK * GROUP) == 0
    n_chunks = per_w // CHUNK
    n_groups = n_chunks // GROUP
    g_rows = GROUP * CHUNK

    mesh = plsc.VectorSubcoreMesh(core_axis_name="c", subcore_axis_name="s")

    @functools.partial(
        pl.kernel,
        mesh=mesh,
        out_type=jax.ShapeDtypeStruct((n_total, D_MODEL), jnp.float32),
        scratch_types=[
            pltpu.VMEM((n_chunks, CHUNK), jnp.int32),
            pltpu.VMEM((NBUF, g_rows, D_MODEL), jnp.float32),
            pltpu.SemaphoreType.DMA,
            pltpu.SemaphoreType.DMA,
        ],
        compiler_params=pltpu.CompilerParams(use_tc_tiling_on_sc=False),
    )
    def gather_kernel(table_hbm, idx_hbm, out_hbm, idx_v, rows_v, gsem, osem):
        wid = lax.axis_index("s") * info.num_cores + lax.axis_index("c")
        base = wid * per_w
        pltpu.sync_copy(idx_hbm.at[pl.ds(wid * n_chunks, n_chunks)], idx_v)

        def fill(g, buf):
            descs = []
            for t in range(GROUP):
                descs.append(pltpu.async_copy(
                    table_hbm.at[idx_v.at[g * GROUP + t]],
                    rows_v.at[buf, pl.ds(t * CHUNK, CHUNK)],
                    gsem))
            for d in descs:
                d.wait()
            pltpu.async_copy(
                rows_v.at[buf],
                out_hbm.at[pl.ds(base + g * g_rows, g_rows)],
                osem)

        def drain_one(buf):
            pltpu.make_async_copy(
                rows_v.at[buf], out_hbm.at[pl.ds(base, g_rows)], osem).wait()

        for g in range(NBUF):
            fill(g, g)

        def body(g, carry):
            buf = lax.rem(g, NBUF)
            drain_one(buf)
            fill(g, buf)
            return carry

        lax.fori_loop(NBUF, n_groups, body, 0)
        for b in range(NBUF):
            drain_one(b)

    return gather_kernel


def kernel(gene_pos, pe):
    b, s = gene_pos.shape
    n_total = b * s
    idx = gene_pos.reshape(n_total // CHUNK, CHUNK)
    table = pe.reshape(pe.shape[0], D_MODEL)
    out = _build_sc_gather(n_total)(table, idx)
    return out.reshape(b, s, D_MODEL)
